# Initial kernel scaffold; baseline (speedup 1.0000x reference)
import jax
import jax.numpy as jnp
from jax.experimental import pallas as pl


def kernel(top_indices, distance_emb):
    raise NotImplementedError("write your pallas kernel here")


# speedup vs baseline: 1.4183x; 1.4183x over previous
import functools

import jax
import jax.numpy as jnp
from jax import lax
from jax.experimental import pallas as pl
from jax.experimental.pallas import tpu as pltpu
from jax.experimental.pallas import tpu_sc as plsc

_N = 16384
_K = 50
_EMB = 64

_NC = 2
_NS = 16
_NW = _NC * _NS
_ROWS_W = _N // _NW
_CH_ROWS = 16
_KP = _K // 2
_CH_P = _CH_ROWS * _KP
_N_CH = _ROWS_W // _CH_ROWS
_GB = 80
_NG = _CH_P // _GB
_L = 16
_NP = _N * _K // 2


def _bucket(d):
    b = jnp.minimum(jnp.maximum(d - 1, 0), 1)
    for thr in (2, 3, 4, 7, 15, 31, 63):
        b = b + jnp.minimum(jnp.maximum(d - thr, 0), 1)
    return b


def _sc_body(tope_hbm, topo_hbm, pt_hbm, out_hbm, te_v, to_v, i_v, o_v, sem):
    wid = lax.axis_index("s") * _NC + lax.axis_index("c")
    lane = lax.iota(jnp.int32, _L)

    def chunk(c, carry):
        row0 = wid * _ROWS_W + c * _CH_ROWS
        p0 = row0 * _KP
        pltpu.sync_copy(tope_hbm.at[pl.ds(p0, _CH_P)], te_v)
        pltpu.sync_copy(topo_hbm.at[pl.ds(p0, _CH_P)], to_v)
        for g in range(_CH_P // _L):
            off = (g * _L) // _KP
            rem = (g * _L) % _KP
            i = row0 + off
            if rem + _L > _KP:
                split = _KP - rem
                i = i + jnp.minimum(jnp.maximum(lane - (split - 1), 0), 1)
            de = jnp.maximum(i - te_v[pl.ds(g * _L, _L)], 1)
            do = jnp.maximum(i - to_v[pl.ds(g * _L, _L)], 1)
            i_v[pl.ds(g * _L, _L)] = _bucket(de) * 9 + _bucket(do)
        cps = [
            pltpu.async_copy(
                pt_hbm.at[i_v.at[pl.ds(gb * _GB, _GB)]],
                o_v.at[pl.ds(gb * _GB, _GB)],
                sem,
            )
            for gb in range(_NG)
        ]
        for cp in cps:
            cp.wait()
        pltpu.sync_copy(o_v, out_hbm.at[pl.ds(p0, _CH_P)])
        return carry

    lax.fori_loop(0, _N_CH, chunk, 0)


@functools.partial(jax.jit)
def _run(top_even, top_odd, pair_table):
    mesh = plsc.VectorSubcoreMesh(core_axis_name="c", subcore_axis_name="s")
    fn = pl.kernel(
        _sc_body,
        mesh=mesh,
        out_type=jax.ShapeDtypeStruct((_NP, 2 * _EMB), jnp.float32),
        scratch_types=[
            pltpu.VMEM((_CH_P,), jnp.int32),
            pltpu.VMEM((_CH_P,), jnp.int32),
            pltpu.VMEM((_CH_P,), jnp.int32),
            pltpu.VMEM((_CH_P, 2 * _EMB), jnp.float32),
            pltpu.SemaphoreType.DMA,
        ],
    )
    return fn(top_even, top_odd, pair_table)


def kernel(top_indices, distance_emb):
    top2 = top_indices.reshape(_NP, 2)
    top_even = top2[:, 0]
    top_odd = top2[:, 1]
    ia, ib = jnp.divmod(jnp.arange(81, dtype=jnp.int32), 9)
    pair_table = jnp.concatenate(
        [distance_emb[ia], distance_emb[ib]], axis=-1)
    pair_table = jnp.pad(pair_table, ((0, 7), (0, 0)))
    out = _run(top_even, top_odd, pair_table)
    return out.reshape(_N, _K, _EMB)


# speedup vs baseline: 8.9046x; 6.2784x over previous
import functools

import jax
import jax.numpy as jnp
from jax import lax
from jax.experimental import pallas as pl
from jax.experimental.pallas import tpu as pltpu
from jax.experimental.pallas import tpu_sc as plsc

_N = 16384
_K = 50
_EMB = 64

_NC = 2
_NS = 16
_NW = _NC * _NS
_ROWS_W = _N // _NW
_CH_ROWS = 16
_KP = _K // 2
_CH_P = _CH_ROWS * _KP
_N_CH = _ROWS_W // _CH_ROWS
_GB = 80
_NG = _CH_P // _GB
_GRP = _GB // 16
_L = 16
_NP = _N * _K // 2


def _bucket(d):
    b = jnp.minimum(jnp.maximum(d - 1, 0), 1)
    for thr in (2, 3, 4, 7, 15, 31, 63):
        b = b + jnp.minimum(jnp.maximum(d - thr, 0), 1)
    return b


def _sc_body(tope_hbm, topo_hbm, pt_hbm, out_hbm,
             te_v, to_v, i_v, o_v0, o_v1, pt_sh, sem_g, sem_w0, sem_w1):
    wid = lax.axis_index("s") * _NC + lax.axis_index("c")
    lane = lax.iota(jnp.int32, _L)

    @pl.when(lax.axis_index("s") == 0)
    def _():
        pltpu.sync_copy(pt_hbm, pt_sh)

    plsc.subcore_barrier()

    def chunk_p0(c):
        return pl.multiple_of((wid * _ROWS_W + c * _CH_ROWS) * _KP, _CH_P)

    def produce(c, o_v):
        row0 = wid * _ROWS_W + c * _CH_ROWS
        p0 = chunk_p0(c)
        pltpu.sync_copy(tope_hbm.at[pl.ds(p0, _CH_P)], te_v)
        pltpu.sync_copy(topo_hbm.at[pl.ds(p0, _CH_P)], to_v)
        cps = []
        for gb in range(_NG):
            for g in range(gb * _GRP, (gb + 1) * _GRP):
                off = (g * _L) // _KP
                rem = (g * _L) % _KP
                i = row0 + off
                if rem + _L > _KP:
                    split = _KP - rem
                    i = i + jnp.minimum(jnp.maximum(lane - (split - 1), 0), 1)
                de = jnp.maximum(i - te_v[pl.ds(g * _L, _L)], 1)
                do = jnp.maximum(i - to_v[pl.ds(g * _L, _L)], 1)
                i_v[pl.ds(g * _L, _L)] = _bucket(de) * 9 + _bucket(do)
            cps.append(pltpu.async_copy(
                pt_sh.at[i_v.at[pl.ds(gb * _GB, _GB)]],
                o_v.at[pl.ds(gb * _GB, _GB)],
                sem_g,
            ))
        for cp in cps:
            cp.wait()

    def wb_start(c, o_v, sem):
        return pltpu.async_copy(
            o_v, out_hbm.at[pl.ds(chunk_p0(c), _CH_P)], sem)

    produce(jnp.int32(0), o_v0)

    def step(c2, carry):
        wb0 = wb_start(2 * c2, o_v0, sem_w0)
        produce(2 * c2 + 1, o_v1)
        wb0.wait()
        wb1 = wb_start(2 * c2 + 1, o_v1, sem_w1)
        produce(2 * c2 + 2, o_v0)
        wb1.wait()
        return carry

    lax.fori_loop(0, (_N_CH - 2) // 2, step, 0)
    wb0 = wb_start(jnp.int32(_N_CH - 2), o_v0, sem_w0)
    produce(jnp.int32(_N_CH - 1), o_v1)
    wb0.wait()
    wb_start(jnp.int32(_N_CH - 1), o_v1, sem_w1).wait()


@functools.partial(jax.jit)
def _run(top_even, top_odd, pair_table):
    mesh = plsc.VectorSubcoreMesh(core_axis_name="c", subcore_axis_name="s")
    fn = pl.kernel(
        _sc_body,
        mesh=mesh,
        out_type=jax.ShapeDtypeStruct((_NP, 2 * _EMB), jnp.float32),
        scratch_types=[
            pltpu.VMEM((_CH_P,), jnp.int32),
            pltpu.VMEM((_CH_P,), jnp.int32),
            pltpu.VMEM((_CH_P,), jnp.int32),
            pltpu.VMEM((_CH_P, 2 * _EMB), jnp.float32),
            pltpu.VMEM((_CH_P, 2 * _EMB), jnp.float32),
            pltpu.VMEM_SHARED((88, 2 * _EMB), jnp.float32),
            pltpu.SemaphoreType.DMA,
            pltpu.SemaphoreType.DMA,
            pltpu.SemaphoreType.DMA,
        ],
    )
    return fn(top_even, top_odd, pair_table)


def kernel(top_indices, distance_emb):
    top2 = top_indices.reshape(_NP, 2)
    top_even = top2[:, 0]
    top_odd = top2[:, 1]
    ia, ib = jnp.divmod(jnp.arange(81, dtype=jnp.int32), 9)
    pair_table = jnp.concatenate(
        [distance_emb[ia], distance_emb[ib]], axis=-1)
    pair_table = jnp.pad(pair_table, ((0, 7), (0, 0)))
    out = _run(top_even, top_odd, pair_table)
    return out.reshape(_N, _K, _EMB)
